# baseline (speedup 1.0000x reference)
import functools

import jax
import jax.numpy as jnp
from jax import lax
from jax.experimental import pallas as pl
from jax.experimental.pallas import tpu as pltpu
from jax.experimental.pallas import tpu_sc as plsc

_NC, _NS = 2, 16
_NW = _NC * _NS
_CHUNK = 128
_DD = 16


_NBUF = 4


def _make_sc_gather(rows_pad: int, d: int):
    chunks_per_w = rows_pad // (_NW * _CHUNK)
    assert chunks_per_w % _NBUF == 0 and chunks_per_w >= 2 * _NBUF
    mesh = plsc.VectorSubcoreMesh(core_axis_name="c", subcore_axis_name="s")

    @functools.partial(
        pl.kernel,
        mesh=mesh,
        out_type=jax.ShapeDtypeStruct((rows_pad, d), jnp.float32),
        scratch_types=[
            pltpu.VMEM((chunks_per_w, _CHUNK), jnp.int32),
            pltpu.VMEM((_NBUF, _CHUNK, d), jnp.float32),
            pltpu.SemaphoreType.DMA((_NBUF,)),
            pltpu.SemaphoreType.DMA((_NBUF,)),
        ],
    )
    def gather_kernel(table_hbm, idx_hbm, out_hbm, idx_v, rows_v, sem_g, sem_s):
        w = lax.axis_index("s") * _NC + (1 - lax.axis_index("c"))
        base = w * chunks_per_w
        pltpu.sync_copy(idx_hbm.at[pl.ds(base, chunks_per_w)], idx_v)

        def gather_copy(k, b):
            return pltpu.make_async_copy(
                table_hbm.at[idx_v.at[k]], rows_v.at[b], sem_g.at[b]
            )

        def store_copy(k, b):
            return pltpu.make_async_copy(
                rows_v.at[b],
                out_hbm.at[pl.ds((base + k) * _CHUNK, _CHUNK)],
                sem_s.at[b],
            )

        for b in range(_NBUF):
            gather_copy(b, b).start()

        def body(j, carry):
            k0 = j * _NBUF
            for b in range(_NBUF):
                gather_copy(k0 + b, b).wait()
                store_copy(k0 + b, b).start()
            for b in range(_NBUF):
                store_copy(k0 + b, b).wait()
                gather_copy(k0 + _NBUF + b, b).start()
            return carry

        lax.fori_loop(0, chunks_per_w // _NBUF - 1, body, 0)

        k0 = chunks_per_w - _NBUF
        for b in range(_NBUF):
            gather_copy(k0 + b, b).wait()
            store_copy(k0 + b, b).start()
        for b in range(_NBUF):
            store_copy(k0 + b, b).wait()

    return gather_kernel


def _tc_body(z_ref, x_ref, a2_ref, er_ref, c_ref, o_ref):
    b, m, d = z_ref.shape
    z3 = z_ref[...]
    a2 = a2_ref[...]
    e = jnp.sum(z3 * a2[None, :, :], axis=-1)
    e = e - jnp.max(e, axis=-1, keepdims=True)
    ex = jnp.exp(e)
    attw = ex / jnp.sum(ex, axis=-1, keepdims=True)
    zf = z3.reshape(b * m, d)
    srep = jnp.dot(zf, er_ref[...], preferred_element_type=jnp.float32,
                   precision=lax.Precision.HIGHEST)
    y3 = (zf * srep).reshape(b, m, d) * attw[:, :, None]
    u = jnp.sum(y3, axis=1) + x_ref[...]
    nrep = jnp.dot(u * u, er_ref[...], preferred_element_type=jnp.float32,
                   precision=lax.Precision.HIGHEST)
    inv = 1.0 / jnp.maximum(jnp.sqrt(nrep), 1e-12)
    o_ref[...] = jnp.where(c_ref[...] > 0.0, u * inv, u)


def kernel(x, neighbors, att, max_iter):
    n, d = x.shape
    m = neighbors.shape[0] // n
    rows = n * m
    gran = _NW * _CHUNK
    rows_pad = ((rows + gran - 1) // gran) * gran

    table = jnp.concatenate([x, jnp.zeros((1, d), x.dtype)], axis=0)
    nb_pad = jnp.concatenate(
        [neighbors, jnp.zeros((rows_pad - rows,), neighbors.dtype)]
    )
    idx2 = nb_pad.reshape(rows_pad // _CHUNK, _CHUNK)

    zg = _make_sc_gather(rows_pad, d)(table, idx2)
    z3 = zg.reshape(rows_pad // m, m, d)

    a2 = att[d:, 0].reshape(1, d)
    ii = lax.broadcasted_iota(jnp.int32, (d, d), 0) // _DD
    jj = lax.broadcasted_iota(jnp.int32, (d, d), 1) // _DD
    erep = (ii == jj).astype(jnp.float32)
    cond = (jnp.asarray(max_iter) > 3).astype(jnp.float32).reshape(1, 1)

    bsz = 200
    out = pl.pallas_call(
        _tc_body,
        grid=(n // bsz,),
        in_specs=[
            pl.BlockSpec((bsz, m, d), lambda i: (i, 0, 0)),
            pl.BlockSpec((bsz, d), lambda i: (i, 0)),
            pl.BlockSpec((1, d), lambda i: (0, 0)),
            pl.BlockSpec((d, d), lambda i: (0, 0)),
            pl.BlockSpec((1, 1), lambda i: (0, 0)),
        ],
        out_specs=pl.BlockSpec((bsz, d), lambda i: (i, 0)),
        out_shape=jax.ShapeDtypeStruct((n, d), jnp.float32),
    )(z3, x, a2, erep, cond)
    return out


# speedup vs baseline: 1.0401x; 1.0401x over previous
import functools

import jax
import jax.numpy as jnp
from jax import lax
from jax.experimental import pallas as pl
from jax.experimental.pallas import tpu as pltpu
from jax.experimental.pallas import tpu_sc as plsc

_NC, _NS = 2, 16
_NW = _NC * _NS
_CHUNK = 128
_DD = 16


_NBUF = 4


def _make_sc_gather(rows_pad: int, d: int):
    chunks_per_w = rows_pad // (_NW * _CHUNK)
    assert chunks_per_w % _NBUF == 0 and chunks_per_w >= 2 * _NBUF
    mesh = plsc.VectorSubcoreMesh(core_axis_name="c", subcore_axis_name="s")

    @functools.partial(
        pl.kernel,
        mesh=mesh,
        out_type=jax.ShapeDtypeStruct((rows_pad, d), jnp.float32),
        scratch_types=[
            pltpu.VMEM((chunks_per_w, _CHUNK), jnp.int32),
            pltpu.VMEM((_NBUF, _CHUNK, d), jnp.float32),
            pltpu.SemaphoreType.DMA((_NBUF,)),
            pltpu.SemaphoreType.DMA((_NBUF,)),
        ],
    )
    def gather_kernel(table_hbm, idx_hbm, out_hbm, idx_v, rows_v, sem_g, sem_s):
        w = lax.axis_index("c") * _NS + lax.axis_index("s")
        base = w * chunks_per_w
        pltpu.sync_copy(idx_hbm.at[pl.ds(base, chunks_per_w)], idx_v)

        def gather_copy(k, b):
            return pltpu.make_async_copy(
                table_hbm.at[idx_v.at[k]], rows_v.at[b], sem_g.at[b]
            )

        def store_copy(k, b):
            return pltpu.make_async_copy(
                rows_v.at[b],
                out_hbm.at[pl.ds((base + k) * _CHUNK, _CHUNK)],
                sem_s.at[b],
            )

        for b in range(_NBUF):
            gather_copy(b, b).start()

        def body(j, carry):
            k0 = j * _NBUF
            for b in range(_NBUF):
                gather_copy(k0 + b, b).wait()
                store_copy(k0 + b, b).start()
            for b in range(_NBUF):
                store_copy(k0 + b, b).wait()
                gather_copy(k0 + _NBUF + b, b).start()
            return carry

        lax.fori_loop(0, chunks_per_w // _NBUF - 1, body, 0)

        k0 = chunks_per_w - _NBUF
        for b in range(_NBUF):
            gather_copy(k0 + b, b).wait()
            store_copy(k0 + b, b).start()
        for b in range(_NBUF):
            store_copy(k0 + b, b).wait()

    return gather_kernel


def _tc_body(z_ref, x_ref, a2_ref, er_ref, c_ref, o_ref):
    b, m, d = z_ref.shape
    z3 = z_ref[...]
    a2 = a2_ref[...]
    e = jnp.sum(z3 * a2[None, :, :], axis=-1)
    e = e - jnp.max(e, axis=-1, keepdims=True)
    ex = jnp.exp(e)
    attw = ex / jnp.sum(ex, axis=-1, keepdims=True)
    zf = z3.reshape(b * m, d)
    srep = jnp.dot(zf, er_ref[...], preferred_element_type=jnp.float32,
                   precision=lax.Precision.HIGHEST)
    y3 = (zf * srep).reshape(b, m, d) * attw[:, :, None]
    u = jnp.sum(y3, axis=1) + x_ref[...]
    nrep = jnp.dot(u * u, er_ref[...], preferred_element_type=jnp.float32,
                   precision=lax.Precision.HIGHEST)
    inv = 1.0 / jnp.maximum(jnp.sqrt(nrep), 1e-12)
    o_ref[...] = jnp.where(c_ref[...] > 0.0, u * inv, u)


def kernel(x, neighbors, att, max_iter):
    n, d = x.shape
    m = neighbors.shape[0] // n
    rows = n * m
    gran = _NW * _CHUNK
    rows_pad = ((rows + gran - 1) // gran) * gran

    table = jnp.concatenate([x, jnp.zeros((1, d), x.dtype)], axis=0)
    nb_pad = jnp.concatenate(
        [neighbors, jnp.zeros((rows_pad - rows,), neighbors.dtype)]
    )
    idx2 = nb_pad.reshape(rows_pad // _CHUNK, _CHUNK)

    zg = _make_sc_gather(rows_pad, d)(table, idx2)
    z3 = zg.reshape(rows_pad // m, m, d)

    a2 = att[d:, 0].reshape(1, d)
    ii = lax.broadcasted_iota(jnp.int32, (d, d), 0) // _DD
    jj = lax.broadcasted_iota(jnp.int32, (d, d), 1) // _DD
    erep = (ii == jj).astype(jnp.float32)
    cond = (jnp.asarray(max_iter) > 3).astype(jnp.float32).reshape(1, 1)

    bsz = 200
    out = pl.pallas_call(
        _tc_body,
        grid=(n // bsz,),
        in_specs=[
            pl.BlockSpec((bsz, m, d), lambda i: (i, 0, 0)),
            pl.BlockSpec((bsz, d), lambda i: (i, 0)),
            pl.BlockSpec((1, d), lambda i: (0, 0)),
            pl.BlockSpec((d, d), lambda i: (0, 0)),
            pl.BlockSpec((1, 1), lambda i: (0, 0)),
        ],
        out_specs=pl.BlockSpec((bsz, d), lambda i: (i, 0)),
        out_shape=jax.ShapeDtypeStruct((n, d), jnp.float32),
    )(z3, x, a2, erep, cond)
    return out


# speedup vs baseline: 2.1420x; 2.0595x over previous
import functools

import jax
import jax.numpy as jnp
from jax import lax
from jax.experimental import pallas as pl
from jax.experimental.pallas import tpu as pltpu
from jax.experimental.pallas import tpu_sc as plsc

_NC, _NS = 2, 16
_NW = _NC * _NS
_CHUNK = 128
_DD = 16


_NBUF = 4


def _make_sc_gather(rows_pad: int, d: int):
    chunks_per_w = rows_pad // (_NW * _CHUNK)
    assert chunks_per_w % _NBUF == 0 and chunks_per_w >= 2 * _NBUF
    mesh = plsc.VectorSubcoreMesh(core_axis_name="c", subcore_axis_name="s")

    @functools.partial(
        pl.kernel,
        mesh=mesh,
        out_type=jax.ShapeDtypeStruct((rows_pad, d), jnp.float32),
        scratch_types=[
            pltpu.VMEM((chunks_per_w, _CHUNK), jnp.int32),
            pltpu.VMEM((_NBUF, _CHUNK, d), jnp.float32),
            pltpu.SemaphoreType.DMA((_NBUF,)),
            pltpu.SemaphoreType.DMA((_NBUF,)),
        ],
    )
    def gather_kernel(table_hbm, idx_hbm, out_hbm, idx_v, rows_v, sem_g, sem_s):
        w = lax.axis_index("c") * _NS + lax.axis_index("s")
        base = w * chunks_per_w
        pltpu.sync_copy(idx_hbm.at[pl.ds(base, chunks_per_w)], idx_v)

        def gather_copy(k, b):
            return pltpu.make_async_copy(
                table_hbm.at[idx_v.at[k]], rows_v.at[b], sem_g.at[b]
            )

        def store_copy(k, b):
            return pltpu.make_async_copy(
                rows_v.at[b],
                out_hbm.at[pl.ds((base + k) * _CHUNK, _CHUNK)],
                sem_s.at[b],
            )

        for b in range(_NBUF):
            gather_copy(b, b).start()

        def body(j, carry):
            k0 = j * _NBUF
            for b in range(_NBUF):
                gather_copy(k0 + b, b).wait()
                store_copy(k0 + b, b).start()
            for b in range(_NBUF):
                store_copy(k0 + b, b).wait()
                gather_copy(k0 + _NBUF + b, b).start()
            return carry

        lax.fori_loop(0, chunks_per_w // _NBUF - 1, body, 0)

        k0 = chunks_per_w - _NBUF
        for b in range(_NBUF):
            gather_copy(k0 + b, b).wait()
            store_copy(k0 + b, b).start()
        for b in range(_NBUF):
            store_copy(k0 + b, b).wait()

    return gather_kernel


def _tc_body(z_ref, x_ref, a2_ref, er_ref, c_ref, o_ref):
    b, m, d = z_ref.shape
    z3 = z_ref[...]
    a2 = a2_ref[...]
    e = jnp.sum(z3 * a2[None, :, :], axis=-1)
    e = e - jnp.max(e, axis=-1, keepdims=True)
    ex = jnp.exp(e)
    attw = ex / jnp.sum(ex, axis=-1, keepdims=True)
    zf = z3.reshape(b * m, d)
    srep = jnp.dot(zf, er_ref[...], preferred_element_type=jnp.float32,
                   precision=lax.Precision.HIGHEST)
    y3 = (zf * srep).reshape(b, m, d) * attw[:, :, None]
    u = jnp.sum(y3, axis=1) + x_ref[...]
    nrep = jnp.dot(u * u, er_ref[...], preferred_element_type=jnp.float32,
                   precision=lax.Precision.HIGHEST)
    inv = 1.0 / jnp.maximum(jnp.sqrt(nrep), 1e-12)
    o_ref[...] = jnp.where(c_ref[...] > 0.0, u * inv, u)


def kernel(x, neighbors, att, max_iter):
    n, d = x.shape
    m = neighbors.shape[0] // n
    rows = n * m
    gran = _NW * _CHUNK
    rows_pad = ((rows + gran - 1) // gran) * gran

    table = jnp.concatenate([x, jnp.zeros((1, d), x.dtype)], axis=0)
    nb_pad = jnp.concatenate(
        [neighbors, jnp.arange(rows_pad - rows, dtype=neighbors.dtype)]
    )
    idx2 = nb_pad.reshape(rows_pad // _CHUNK, _CHUNK)

    zg = _make_sc_gather(rows_pad, d)(table, idx2)
    z3 = zg.reshape(rows_pad // m, m, d)

    a2 = att[d:, 0].reshape(1, d)
    ii = lax.broadcasted_iota(jnp.int32, (d, d), 0) // _DD
    jj = lax.broadcasted_iota(jnp.int32, (d, d), 1) // _DD
    erep = (ii == jj).astype(jnp.float32)
    cond = (jnp.asarray(max_iter) > 3).astype(jnp.float32).reshape(1, 1)

    bsz = 200
    out = pl.pallas_call(
        _tc_body,
        grid=(n // bsz,),
        in_specs=[
            pl.BlockSpec((bsz, m, d), lambda i: (i, 0, 0)),
            pl.BlockSpec((bsz, d), lambda i: (i, 0)),
            pl.BlockSpec((1, d), lambda i: (0, 0)),
            pl.BlockSpec((d, d), lambda i: (0, 0)),
            pl.BlockSpec((1, 1), lambda i: (0, 0)),
        ],
        out_specs=pl.BlockSpec((bsz, d), lambda i: (i, 0)),
        out_shape=jax.ShapeDtypeStruct((n, d), jnp.float32),
    )(z3, x, a2, erep, cond)
    return out


# speedup vs baseline: 2.7661x; 1.2914x over previous
import functools

import jax
import jax.numpy as jnp
from jax import lax
from jax.experimental import pallas as pl
from jax.experimental.pallas import tpu as pltpu
from jax.experimental.pallas import tpu_sc as plsc

_NC, _NS = 2, 16
_NW = _NC * _NS
_CHUNK = 128
_DD = 16


_NBUF = 4


def _make_sc_gather(rows_pad: int, d: int):
    chunks_per_w = rows_pad // (_NW * _CHUNK)
    assert chunks_per_w % _NBUF == 0 and chunks_per_w >= 2 * _NBUF
    mesh = plsc.VectorSubcoreMesh(core_axis_name="c", subcore_axis_name="s")

    @functools.partial(
        pl.kernel,
        mesh=mesh,
        out_type=jax.ShapeDtypeStruct((rows_pad, d), jnp.float32),
        scratch_types=[
            pltpu.VMEM((chunks_per_w, _CHUNK), jnp.int32),
            pltpu.VMEM((_NBUF, _CHUNK, d), jnp.float32),
            pltpu.SemaphoreType.DMA((_NBUF,)),
            pltpu.SemaphoreType.DMA((_NBUF,)),
        ],
    )
    def gather_kernel(table_hbm, idx_hbm, out_hbm, idx_v, rows_v, sem_g, sem_s):
        w = lax.axis_index("c") * _NS + lax.axis_index("s")
        base = w * chunks_per_w
        pltpu.sync_copy(idx_hbm.at[pl.ds(base, chunks_per_w)], idx_v)

        def gather_copy(k, b):
            return pltpu.make_async_copy(
                table_hbm.at[idx_v.at[k]], rows_v.at[b], sem_g.at[b]
            )

        def store_copy(k, b):
            return pltpu.make_async_copy(
                rows_v.at[b],
                out_hbm.at[pl.ds((base + k) * _CHUNK, _CHUNK)],
                sem_s.at[b],
            )

        for b in range(_NBUF):
            gather_copy(b, b).start()

        def body(j, carry):
            k0 = j * _NBUF
            for b in range(_NBUF):
                gather_copy(k0 + b, b).wait()
                store_copy(k0 + b, b).start()
            for b in range(_NBUF):
                store_copy(k0 + b, b).wait()
                gather_copy(k0 + _NBUF + b, b).start()
            return carry

        lax.fori_loop(0, chunks_per_w // _NBUF - 1, body, 0)

        k0 = chunks_per_w - _NBUF
        for b in range(_NBUF):
            gather_copy(k0 + b, b).wait()
            store_copy(k0 + b, b).start()
        for b in range(_NBUF):
            store_copy(k0 + b, b).wait()

    return gather_kernel


def _tc_body(z_ref, x_ref, a2_ref, er_ref, c_ref, o_ref):
    b, m, d = z_ref.shape
    z3 = z_ref[...]
    zf = z3.reshape(b * m, d)
    ecol = jnp.dot(zf, a2_ref[...], preferred_element_type=jnp.float32)
    e3 = ecol.reshape(b, m, 1)
    e3 = e3 - jnp.max(e3, axis=1, keepdims=True)
    ex = jnp.exp(e3)
    attw3 = ex / jnp.sum(ex, axis=1, keepdims=True)
    srep = jnp.dot(zf, er_ref[...], preferred_element_type=jnp.float32)
    y3 = (zf * srep).reshape(b, m, d) * attw3
    u = jnp.sum(y3, axis=1) + x_ref[...]
    nrep = jnp.dot(u * u, er_ref[...], preferred_element_type=jnp.float32)
    inv = 1.0 / jnp.maximum(jnp.sqrt(nrep), 1e-12)
    o_ref[...] = jnp.where(c_ref[...] > 0.0, u * inv, u)


def kernel(x, neighbors, att, max_iter):
    n, d = x.shape
    m = neighbors.shape[0] // n
    rows = n * m
    gran = _NW * _CHUNK
    rows_pad = ((rows + gran - 1) // gran) * gran

    table = jnp.concatenate([x, jnp.zeros((1, d), x.dtype)], axis=0)
    nb_pad = jnp.concatenate(
        [neighbors, jnp.arange(rows_pad - rows, dtype=neighbors.dtype)]
    )
    idx2 = nb_pad.reshape(rows_pad // _CHUNK, _CHUNK)

    zg = _make_sc_gather(rows_pad, d)(table, idx2)
    z3 = zg.reshape(rows_pad // m, m, d)

    a2 = att[d:, :]
    ii = lax.broadcasted_iota(jnp.int32, (d, d), 0) // _DD
    jj = lax.broadcasted_iota(jnp.int32, (d, d), 1) // _DD
    erep = (ii == jj).astype(jnp.float32)
    cond = (jnp.asarray(max_iter) > 3).astype(jnp.float32).reshape(1, 1)

    bsz = 200
    out = pl.pallas_call(
        _tc_body,
        grid=(n // bsz,),
        in_specs=[
            pl.BlockSpec((bsz, m, d), lambda i: (i, 0, 0)),
            pl.BlockSpec((bsz, d), lambda i: (i, 0)),
            pl.BlockSpec((d, 1), lambda i: (0, 0)),
            pl.BlockSpec((d, d), lambda i: (0, 0)),
            pl.BlockSpec((1, 1), lambda i: (0, 0)),
        ],
        out_specs=pl.BlockSpec((bsz, d), lambda i: (i, 0)),
        out_shape=jax.ShapeDtypeStruct((n, d), jnp.float32),
    )(z3, x, a2, erep, cond)
    return out


# speedup vs baseline: 3.0162x; 1.0904x over previous
import functools

import jax
import jax.numpy as jnp
from jax import lax
from jax.experimental import pallas as pl
from jax.experimental.pallas import tpu as pltpu
from jax.experimental.pallas import tpu_sc as plsc

_NC, _NS = 2, 16
_NW = _NC * _NS
_CHUNK = 128
_DD = 16


_NBUF = 4


def _make_sc_gather(rows_pad: int, d: int):
    chunks_per_w = rows_pad // (_NW * _CHUNK)
    assert chunks_per_w % _NBUF == 0 and chunks_per_w >= 2 * _NBUF
    mesh = plsc.VectorSubcoreMesh(core_axis_name="c", subcore_axis_name="s")

    @functools.partial(
        pl.kernel,
        mesh=mesh,
        out_type=jax.ShapeDtypeStruct((rows_pad, d), jnp.float32),
        scratch_types=[
            pltpu.VMEM((chunks_per_w, _CHUNK), jnp.int32),
            pltpu.VMEM((_NBUF, _CHUNK, d), jnp.float32),
            pltpu.SemaphoreType.DMA((_NBUF,)),
            pltpu.SemaphoreType.DMA((_NBUF,)),
        ],
    )
    def gather_kernel(table_hbm, idx_hbm, out_hbm, idx_v, rows_v, sem_g, sem_s):
        w = lax.axis_index("c") * _NS + lax.axis_index("s")
        base = w * chunks_per_w
        pltpu.sync_copy(idx_hbm.at[pl.ds(base, chunks_per_w)], idx_v)

        def gather_copy(k, b):
            return pltpu.make_async_copy(
                table_hbm.at[idx_v.at[k]], rows_v.at[b], sem_g.at[b]
            )

        def store_copy(k, b):
            return pltpu.make_async_copy(
                rows_v.at[b],
                out_hbm.at[pl.ds((base + k) * _CHUNK, _CHUNK)],
                sem_s.at[b],
            )

        for b in range(_NBUF):
            gather_copy(b, b).start()

        def body(j, carry):
            k0 = j * _NBUF
            for b in range(_NBUF):
                gather_copy(k0 + b, b).wait()
                store_copy(k0 + b, b).start()
            for b in range(_NBUF):
                store_copy(k0 + b, b).wait()
                gather_copy(k0 + _NBUF + b, b).start()
            return carry

        lax.fori_loop(0, chunks_per_w // _NBUF - 1, body, 0)

        k0 = chunks_per_w - _NBUF
        for b in range(_NBUF):
            gather_copy(k0 + b, b).wait()
            store_copy(k0 + b, b).start()
        for b in range(_NBUF):
            store_copy(k0 + b, b).wait()

    return gather_kernel


def _tc_body(z_ref, x_ref, a2_ref, er_ref, c_ref, o_ref):
    b, m, d = z_ref.shape
    z3 = z_ref[...]
    zf = z3.reshape(b * m, d)
    ecol = jnp.dot(zf, a2_ref[...], preferred_element_type=jnp.float32)
    e3 = ecol.reshape(b, m, 1)
    e3 = e3 - jnp.max(e3, axis=1, keepdims=True)
    ex = jnp.exp(e3)
    attw3 = ex / jnp.sum(ex, axis=1, keepdims=True)
    srep = jnp.dot(zf, er_ref[...], preferred_element_type=jnp.float32)
    y3 = (zf * srep).reshape(b, m, d) * attw3
    u = jnp.sum(y3, axis=1) + x_ref[...]
    nrep = jnp.dot(u * u, er_ref[...], preferred_element_type=jnp.float32)
    inv = 1.0 / jnp.maximum(jnp.sqrt(nrep), 1e-12)
    o_ref[...] = jnp.where(c_ref[...] > 0.0, u * inv, u)


def kernel(x, neighbors, att, max_iter):
    n, d = x.shape
    m = neighbors.shape[0] // n
    gran = _NW * _CHUNK

    table = jnp.concatenate([x, jnp.zeros((1, d), x.dtype)], axis=0)
    a2 = att[d:, :]
    ii = lax.broadcasted_iota(jnp.int32, (d, d), 0) // _DD
    jj = lax.broadcasted_iota(jnp.int32, (d, d), 1) // _DD
    erep = (ii == jj).astype(jnp.float32)
    cond = (jnp.asarray(max_iter) > 3).astype(jnp.float32).reshape(1, 1)

    phases = 5
    bsz = 200
    nodes_p = n // phases
    rows_p = nodes_p * m
    rows_pad_p = ((rows_p + gran - 1) // gran) * gran
    pad_idx = jnp.arange(rows_pad_p - rows_p, dtype=neighbors.dtype)

    gather_fn = _make_sc_gather(rows_pad_p, d)
    blocks_p = nodes_p // bsz

    def tc_call(z3_p, p):
        return pl.pallas_call(
            _tc_body,
            grid=(blocks_p,),
            in_specs=[
                pl.BlockSpec((bsz, m, d), lambda i: (i, 0, 0)),
                pl.BlockSpec((bsz, d), lambda i, p=p: (p * blocks_p + i, 0)),
                pl.BlockSpec((d, 1), lambda i: (0, 0)),
                pl.BlockSpec((d, d), lambda i: (0, 0)),
                pl.BlockSpec((1, 1), lambda i: (0, 0)),
            ],
            out_specs=pl.BlockSpec((bsz, d), lambda i: (i, 0)),
            out_shape=jax.ShapeDtypeStruct((nodes_p, d), jnp.float32),
        )(z3_p, x, a2, erep, cond)

    outs = []
    for p in range(phases):
        nb_p = lax.dynamic_slice_in_dim(neighbors, p * rows_p, rows_p)
        idx2 = jnp.concatenate([nb_p, pad_idx]).reshape(
            rows_pad_p // _CHUNK, _CHUNK
        )
        zg = gather_fn(table, idx2)
        z3_p = zg.reshape(rows_pad_p // m, m, d)
        outs.append(tc_call(z3_p, p))
    return jnp.concatenate(outs, axis=0)


# speedup vs baseline: 3.1901x; 1.0577x over previous
import functools

import jax
import jax.numpy as jnp
from jax import lax
from jax.experimental import pallas as pl
from jax.experimental.pallas import tpu as pltpu
from jax.experimental.pallas import tpu_sc as plsc

_NC, _NS = 2, 16
_NW = _NC * _NS
_CHUNK = 128
_DD = 16


_NBUF = 6


def _make_sc_gather(rows_pad: int, d: int):
    chunks_per_w = rows_pad // (_NW * _CHUNK)
    nbuf = min(_NBUF, chunks_per_w)
    mesh = plsc.VectorSubcoreMesh(core_axis_name="c", subcore_axis_name="s")

    @functools.partial(
        pl.kernel,
        mesh=mesh,
        out_type=jax.ShapeDtypeStruct((rows_pad, d), jnp.float32),
        scratch_types=[
            pltpu.VMEM((chunks_per_w, _CHUNK), jnp.int32),
            pltpu.VMEM((nbuf, _CHUNK, d), jnp.float32),
            pltpu.SemaphoreType.DMA((nbuf,)),
            pltpu.SemaphoreType.DMA((nbuf,)),
        ],
    )
    def gather_kernel(table_hbm, idx_hbm, out_hbm, idx_v, rows_v, sem_g, sem_s):
        w = lax.axis_index("c") * _NS + lax.axis_index("s")
        base = w * chunks_per_w
        pltpu.sync_copy(idx_hbm.at[pl.ds(base, chunks_per_w)], idx_v)

        def gather_copy(k, b):
            return pltpu.make_async_copy(
                table_hbm.at[idx_v.at[k]], rows_v.at[b], sem_g.at[b]
            )

        def store_copy(k, b):
            return pltpu.make_async_copy(
                rows_v.at[b],
                out_hbm.at[pl.ds((base + k) * _CHUNK, _CHUNK)],
                sem_s.at[b],
            )

        for k in range(nbuf):
            gather_copy(k, k).start()
        store_waited = set()
        for k in range(chunks_per_w):
            gather_copy(k, k % nbuf).wait()
            store_copy(k, k % nbuf).start()
            pk = k - 1
            if pk >= 0 and pk + nbuf < chunks_per_w:
                store_copy(pk, pk % nbuf).wait()
                store_waited.add(pk)
                gather_copy(pk + nbuf, pk % nbuf).start()
        for k in range(chunks_per_w):
            if k not in store_waited:
                store_copy(k, k % nbuf).wait()

    return gather_kernel


def _tc_body(z_ref, x_ref, a2_ref, er_ref, c_ref, o_ref):
    b, m, d = z_ref.shape
    z3 = z_ref[...]
    zf = z3.reshape(b * m, d)
    ecol = jnp.dot(zf, a2_ref[...], preferred_element_type=jnp.float32)
    e3 = ecol.reshape(b, m, 1)
    e3 = e3 - jnp.max(e3, axis=1, keepdims=True)
    ex = jnp.exp(e3)
    attw3 = ex / jnp.sum(ex, axis=1, keepdims=True)
    srep = jnp.dot(zf, er_ref[...], preferred_element_type=jnp.float32)
    y3 = (zf * srep).reshape(b, m, d) * attw3
    u = jnp.sum(y3, axis=1) + x_ref[...]
    nrep = jnp.dot(u * u, er_ref[...], preferred_element_type=jnp.float32)
    inv = 1.0 / jnp.maximum(jnp.sqrt(nrep), 1e-12)
    o_ref[...] = jnp.where(c_ref[...] > 0.0, u * inv, u)


def kernel(x, neighbors, att, max_iter):
    n, d = x.shape
    m = neighbors.shape[0] // n
    gran = _NW * _CHUNK

    table = jnp.concatenate([x, jnp.zeros((1, d), x.dtype)], axis=0)
    a2 = att[d:, :]
    ii = lax.broadcasted_iota(jnp.int32, (d, d), 0) // _DD
    jj = lax.broadcasted_iota(jnp.int32, (d, d), 1) // _DD
    erep = (ii == jj).astype(jnp.float32)
    cond = (jnp.asarray(max_iter) > 3).astype(jnp.float32).reshape(1, 1)

    phases = 5
    bsz = 400
    nodes_p = n // phases
    rows_p = nodes_p * m
    rows_pad_p = ((rows_p + gran - 1) // gran) * gran
    pad_idx = jnp.arange(rows_pad_p - rows_p, dtype=neighbors.dtype)

    gather_fn = _make_sc_gather(rows_pad_p, d)
    blocks_p = nodes_p // bsz

    def tc_call(z3_p, p):
        return pl.pallas_call(
            _tc_body,
            grid=(blocks_p,),
            in_specs=[
                pl.BlockSpec((bsz, m, d), lambda i: (i, 0, 0)),
                pl.BlockSpec((bsz, d), lambda i, p=p: (p * blocks_p + i, 0)),
                pl.BlockSpec((d, 1), lambda i: (0, 0)),
                pl.BlockSpec((d, d), lambda i: (0, 0)),
                pl.BlockSpec((1, 1), lambda i: (0, 0)),
            ],
            out_specs=pl.BlockSpec((bsz, d), lambda i: (i, 0)),
            out_shape=jax.ShapeDtypeStruct((nodes_p, d), jnp.float32),
        )(z3_p, x, a2, erep, cond)

    outs = []
    for p in range(phases):
        nb_p = lax.dynamic_slice_in_dim(neighbors, p * rows_p, rows_p)
        idx2 = jnp.concatenate([nb_p, pad_idx]).reshape(
            rows_pad_p // _CHUNK, _CHUNK
        )
        zg = gather_fn(table, idx2)
        z3_p = zg.reshape(rows_pad_p // m, m, d)
        outs.append(tc_call(z3_p, p))
    return jnp.concatenate(outs, axis=0)
